# baseline (speedup 1.0000x reference)
import functools

import jax
import jax.numpy as jnp
from jax import lax
from jax.experimental import pallas as pl
from jax.experimental.pallas import tpu as pltpu
from jax.experimental.pallas import tpu_sc as plsc

NC = 2
NS = 16
L = 16
NW = NC * NS


def _sc_mesh():
    return plsc.VectorSubcoreMesh(
        core_axis_name="c", subcore_axis_name="s", num_cores=NC, num_subcores=NS
    )


def _deg_kernel(R, G, n_pad):
    CH = R // (NW * G)
    nr = n_pad // L

    @functools.partial(
        pl.kernel,
        mesh=_sc_mesh(),
        out_type=jax.ShapeDtypeStruct((NW, nr, L), jnp.float32),
        compiler_params=pltpu.CompilerParams(needs_layout_passes=False, use_tc_tiling_on_sc=False),
        scratch_types=[
            pltpu.VMEM((G, 128), jnp.int32),
            pltpu.VMEM((G, 128), jnp.int32),
            pltpu.VMEM((G, 128), jnp.float32),
            pltpu.VMEM((nr, L), jnp.float32),
        ],
    )
    def k(src_h, dst_h, w_h, degp_h, sv, dv, wv, deg_l):
        cid = lax.axis_index("c")
        sid = lax.axis_index("s")
        wid = sid * NC + cid
        zeros = jnp.zeros((L,), jnp.float32)

        def zb(i, carry):
            deg_l[i, :] = zeros
            return carry

        lax.fori_loop(0, nr, zb, 0)

        base = wid * (G * CH)

        def chunk(kc, carry):
            r0 = pl.multiple_of(base + kc * G, 8)
            pltpu.sync_copy(src_h.at[pl.ds(r0, G)], sv)
            pltpu.sync_copy(dst_h.at[pl.ds(r0, G)], dv)
            pltpu.sync_copy(w_h.at[pl.ds(r0, G)], wv)
            for j in range(G * (128 // L)):
                jr, jc = divmod(j, 128 // L)
                s16 = sv[jr, pl.ds(jc * L, L)]
                d16 = dv[jr, pl.ds(jc * L, L)]
                w16 = wv[jr, pl.ds(jc * L, L)]
                row = lax.shift_right_logical(s16, 4)
                col = jnp.bitwise_and(s16, 15)
                plsc.addupdate_scatter(
                    deg_l, [row, col], jnp.where(s16 != d16, w16, 0.0)
                )
            return carry

        lax.fori_loop(0, CH, chunk, 0)
        pltpu.sync_copy(deg_l, degp_h.at[wid])

    return k


def _main_kernel(R, G, n_pad):
    CH = R // (NW * G)
    rpt = n_pad // NS
    ZR = 512
    nfull = rpt // ZR
    rem = rpt % ZR

    @functools.partial(
        pl.kernel,
        mesh=_sc_mesh(),
        out_type=jax.ShapeDtypeStruct((NC, n_pad, L), jnp.float32),
        compiler_params=pltpu.CompilerParams(needs_layout_passes=False, use_tc_tiling_on_sc=False),
        scratch_types=[
            pltpu.VMEM((G, 128), jnp.int32),
            pltpu.VMEM((G, 128), jnp.int32),
            pltpu.VMEM((G, 128), jnp.float32),
            pltpu.VMEM((128, L), jnp.float32),
            pltpu.VMEM((ZR, L), jnp.float32),
            pltpu.VMEM_SHARED((n_pad, L), jnp.float32),
            pltpu.SemaphoreType.DMA,
            pltpu.SemaphoreType.DMA,
        ],
    )
    def k(src_h, dst_h, w_h, xt_h, accp_h, sv, dv, wv, rows, zbuf, acc_sh,
          gsem, ssem):
        cid = lax.axis_index("c")
        sid = lax.axis_index("s")
        wid = sid * NC + cid
        zeros = jnp.zeros((L,), jnp.float32)

        def z1(i, carry):
            zbuf[i, :] = zeros
            return carry

        lax.fori_loop(0, ZR, z1, 0)

        tbase = sid * rpt
        for i in range(nfull):
            pltpu.sync_copy(
                zbuf, acc_sh.at[pl.ds(pl.multiple_of(tbase + i * ZR, 16), ZR)]
            )
        if rem:
            pltpu.sync_copy(
                zbuf.at[pl.ds(0, rem)],
                acc_sh.at[pl.ds(pl.multiple_of(tbase + nfull * ZR, 16), rem)],
            )
        plsc.subcore_barrier()

        base = wid * (G * CH)

        def chunk(kc, carry):
            r0 = pl.multiple_of(base + kc * G, 8)
            pltpu.sync_copy(src_h.at[pl.ds(r0, G)], sv)
            pltpu.sync_copy(dst_h.at[pl.ds(r0, G)], dv)
            pltpu.sync_copy(w_h.at[pl.ds(r0, G)], wv)
            for jr in range(G):
                pltpu.async_copy(xt_h.at[sv.at[jr]], rows, gsem).wait()
                for jc in range(128 // L):
                    s16 = sv[jr, pl.ds(jc * L, L)]
                    d16 = dv[jr, pl.ds(jc * L, L)]
                    w16 = wv[jr, pl.ds(jc * L, L)]
                    wp = jnp.where(s16 != d16, w16, 0.0)
                    eidx = lax.iota(jnp.int32, L) + jc * L
                    for col in range(L):
                        csplat = jnp.full((L,), col, jnp.int32)
                        vals = plsc.load_gather(rows, [eidx, csplat])
                        plsc.store_scatter(rows, [eidx, csplat], vals * wp)
                pltpu.async_copy(
                    rows, acc_sh.at[dv.at[jr]], ssem, add=True
                ).wait()
            return carry

        lax.fori_loop(0, CH, chunk, 0)
        plsc.subcore_barrier()
        r0 = pl.multiple_of(sid * rpt, 16)
        pltpu.sync_copy(
            acc_sh.at[pl.ds(r0, rpt)], accp_h.at[cid, pl.ds(r0, rpt)]
        )

    return k


def _tc_mid(n_pad, nw):
    B = 2048

    def body(degp_ref, x_ref, dis_ref, xt_ref):
        deg = jnp.sum(degp_ref[...], axis=0)
        dis = jnp.where(deg > 0, lax.rsqrt(jnp.maximum(deg, 1e-12)), 0.0)
        dis_ref[...] = dis
        xt_ref[...] = dis[:, None] * x_ref[...]

    return pl.pallas_call(
        body,
        grid=(pl.cdiv(n_pad, B),),
        in_specs=[
            pl.BlockSpec((nw, B), lambda i: (0, i)),
            pl.BlockSpec((B, L), lambda i: (i, 0)),
        ],
        out_specs=[
            pl.BlockSpec((B,), lambda i: (i,)),
            pl.BlockSpec((B, L), lambda i: (i, 0)),
        ],
        out_shape=[
            jax.ShapeDtypeStruct((n_pad,), jnp.float32),
            jax.ShapeDtypeStruct((n_pad, L), jnp.float32),
        ],
    )


def _tc_head(n_pad, F):
    B = 2048

    def body(x_ref, accp_ref, dis_ref, wz0, wz1, wh0, wh1, bz, bh, wl, blr, out_ref):
        x = x_ref[...]
        tx1 = -dis_ref[...][:, None] * (accp_ref[0] + accp_ref[1])
        z = jax.nn.sigmoid(
            jnp.dot(x, wz0[...], preferred_element_type=jnp.float32)
            + jnp.dot(tx1, wz1[...], preferred_element_type=jnp.float32)
            + bz[...]
        )
        ht = jnp.tanh(
            jnp.dot(x, wh0[...], preferred_element_type=jnp.float32)
            + jnp.dot(tx1, wh1[...], preferred_element_type=jnp.float32)
            + bh[...]
        )
        h = jax.nn.relu((1.0 - z) * ht)
        o = jnp.dot(h, wl[...], preferred_element_type=jnp.float32) + blr[...]
        out_ref[...] = o[:, 0]

    seq = L
    return pl.pallas_call(
        body,
        grid=(pl.cdiv(n_pad, B),),
        in_specs=[
            pl.BlockSpec((B, seq), lambda i: (i, 0)),
            pl.BlockSpec((NC, B, seq), lambda i: (0, i, 0)),
            pl.BlockSpec((B,), lambda i: (i,)),
            pl.BlockSpec((seq, F), lambda i: (0, 0)),
            pl.BlockSpec((seq, F), lambda i: (0, 0)),
            pl.BlockSpec((seq, F), lambda i: (0, 0)),
            pl.BlockSpec((seq, F), lambda i: (0, 0)),
            pl.BlockSpec((F,), lambda i: (0,)),
            pl.BlockSpec((F,), lambda i: (0,)),
            pl.BlockSpec((F, 1), lambda i: (0, 0)),
            pl.BlockSpec((1, 1), lambda i: (0, 0)),
        ],
        out_specs=pl.BlockSpec((B,), lambda i: (i,)),
        out_shape=jax.ShapeDtypeStruct((n_pad,), jnp.float32),
    )


def kernel(y, edge_index, edge_attr, Wxz0, Wxz1, bxz, Whz0, Whz1, bhz,
           Wxr0, Wxr1, bxr, Whr0, Whr1, bhr, Wxh0, Wxh1, bxh, Whh0, Whh1, bhh,
           Wl, bl):
    n = y.shape[1]
    e = edge_index.shape[1]
    seq = y.shape[0] - 1
    F = Wxz0.shape[1]
    G = 8
    unit = NW * G * 128 * 2
    epad = ((e + unit - 1) // unit) * unit
    R = epad // 128
    n_pad = ((n + 127) // 128) * 128

    src = jnp.pad(edge_index[0], (0, epad - e)).reshape(R, 128)
    dst = jnp.pad(edge_index[1], (0, epad - e)).reshape(R, 128)
    w = jnp.pad(edge_attr[:, 0], (0, epad - e)).reshape(R, 128)
    x = jnp.pad(y[:seq].T, ((0, n_pad - n), (0, 0)))

    degp3 = _deg_kernel(R, G, n_pad)(src, dst, w)
    degp = degp3.reshape(NW, n_pad)
    dis, xt = _tc_mid(n_pad, NW)(degp, x)
    accp = _main_kernel(R, G, n_pad)(src, dst, w, xt)
    out = _tc_head(n_pad, F)(
        x, accp, dis, Wxz0, Wxz1, Wxh0, Wxh1, bxz + bhz, bxh + bhh, Wl,
        bl.reshape(1, 1)
    )
    return out[:n]


# speedup vs baseline: 1.2500x; 1.2500x over previous
import functools

import jax
import jax.numpy as jnp
from jax import lax
from jax.experimental import pallas as pl
from jax.experimental.pallas import tpu as pltpu
from jax.experimental.pallas import tpu_sc as plsc

NC = 2
NS = 16
L = 16
NW = NC * NS


def _sc_mesh():
    return plsc.VectorSubcoreMesh(
        core_axis_name="c", subcore_axis_name="s", num_cores=NC, num_subcores=NS
    )


def _deg_kernel(R, G, n_pad):
    CH = R // (NW * G)
    nr = n_pad // L

    @functools.partial(
        pl.kernel,
        mesh=_sc_mesh(),
        out_type=jax.ShapeDtypeStruct((NW, nr, L), jnp.float32),
        compiler_params=pltpu.CompilerParams(needs_layout_passes=False, use_tc_tiling_on_sc=False),
        scratch_types=[
            pltpu.VMEM((G, 128), jnp.int32),
            pltpu.VMEM((G, 128), jnp.int32),
            pltpu.VMEM((G, 128), jnp.float32),
            pltpu.VMEM((nr, L), jnp.float32),
        ],
    )
    def k(src_h, dst_h, w_h, degp_h, sv, dv, wv, deg_l):
        cid = lax.axis_index("c")
        sid = lax.axis_index("s")
        wid = sid * NC + cid
        zeros = jnp.zeros((L,), jnp.float32)

        def zb(i, carry):
            deg_l[i, :] = zeros
            return carry

        lax.fori_loop(0, nr, zb, 0)

        base = wid * (G * CH)

        def chunk(kc, carry):
            r0 = pl.multiple_of(base + kc * G, 8)
            pltpu.sync_copy(src_h.at[pl.ds(r0, G)], sv)
            pltpu.sync_copy(dst_h.at[pl.ds(r0, G)], dv)
            pltpu.sync_copy(w_h.at[pl.ds(r0, G)], wv)
            for j in range(G * (128 // L)):
                jr, jc = divmod(j, 128 // L)
                s16 = sv[jr, pl.ds(jc * L, L)]
                d16 = dv[jr, pl.ds(jc * L, L)]
                w16 = wv[jr, pl.ds(jc * L, L)]
                row = lax.shift_right_logical(s16, 4)
                col = jnp.bitwise_and(s16, 15)
                plsc.addupdate_scatter(
                    deg_l, [row, col], jnp.where(s16 != d16, w16, 0.0)
                )
            return carry

        lax.fori_loop(0, CH, chunk, 0)
        pltpu.sync_copy(deg_l, degp_h.at[wid])

    return k


def _main_kernel(R, G, n_pad):
    CH = R // (NW * G)
    rpt = n_pad // NS
    ZR = 512
    nfull = rpt // ZR
    rem = rpt % ZR

    @functools.partial(
        pl.kernel,
        mesh=_sc_mesh(),
        out_type=jax.ShapeDtypeStruct((NC, n_pad, L), jnp.float32),
        compiler_params=pltpu.CompilerParams(needs_layout_passes=False, use_tc_tiling_on_sc=False),
        scratch_types=[
            pltpu.VMEM((G, 128), jnp.int32),
            pltpu.VMEM((G, 128), jnp.int32),
            pltpu.VMEM((G, 128), jnp.float32),
            pltpu.VMEM((2, 128, L), jnp.float32),
            pltpu.VMEM((ZR, L), jnp.float32),
            pltpu.VMEM_SHARED((n_pad, L), jnp.float32),
            pltpu.SemaphoreType.DMA,
            pltpu.SemaphoreType.DMA,
        ],
    )
    def k(src_h, dst_h, w_h, xt_h, accp_h, sv, dv, wv, rows, zbuf, acc_sh,
          gsem, ssem):
        cid = lax.axis_index("c")
        sid = lax.axis_index("s")
        wid = sid * NC + cid
        zeros = jnp.zeros((L,), jnp.float32)

        def z1(i, carry):
            zbuf[i, :] = zeros
            return carry

        lax.fori_loop(0, ZR, z1, 0)

        tbase = sid * rpt
        for i in range(nfull):
            pltpu.sync_copy(
                zbuf, acc_sh.at[pl.ds(pl.multiple_of(tbase + i * ZR, 16), ZR)]
            )
        if rem:
            pltpu.sync_copy(
                zbuf.at[pl.ds(0, rem)],
                acc_sh.at[pl.ds(pl.multiple_of(tbase + nfull * ZR, 16), rem)],
            )
        plsc.subcore_barrier()

        base = wid * (G * CH)

        def scale(b, jr):
            for jc in range(128 // L):
                s16 = sv[jr, pl.ds(jc * L, L)]
                d16 = dv[jr, pl.ds(jc * L, L)]
                w16 = wv[jr, pl.ds(jc * L, L)]
                wp = jnp.where(s16 != d16, w16, 0.0)
                eidx = lax.iota(jnp.int32, L) + jc * L
                for col in range(L):
                    csplat = jnp.full((L,), col, jnp.int32)
                    vals = plsc.load_gather(rows.at[b], [eidx, csplat])
                    plsc.store_scatter(rows.at[b], [eidx, csplat], vals * wp)

        def chunk(kc, carry):
            r0 = pl.multiple_of(base + kc * G, 8)
            pltpu.sync_copy(src_h.at[pl.ds(r0, G)], sv)
            pltpu.sync_copy(dst_h.at[pl.ds(r0, G)], dv)
            pltpu.sync_copy(w_h.at[pl.ds(r0, G)], wv)
            gh = [None] * G
            sh = [None] * G
            gh[0] = pltpu.async_copy(xt_h.at[sv.at[0]], rows.at[0], gsem)
            for jr in range(G):
                b = jr & 1
                if jr >= 1:
                    sh[jr - 1].wait()
                gh[jr].wait()
                if jr + 1 < G:
                    gh[jr + 1] = pltpu.async_copy(
                        xt_h.at[sv.at[jr + 1]], rows.at[1 - b], gsem
                    )
                scale(b, jr)
                sh[jr] = pltpu.async_copy(
                    rows.at[b], acc_sh.at[dv.at[jr]], ssem, add=True
                )
            sh[G - 1].wait()
            return carry

        lax.fori_loop(0, CH, chunk, 0)
        plsc.subcore_barrier()
        r0 = pl.multiple_of(sid * rpt, 16)
        pltpu.sync_copy(
            acc_sh.at[pl.ds(r0, rpt)], accp_h.at[cid, pl.ds(r0, rpt)]
        )

    return k


def _tc_mid(n_pad, nw):
    B = 2048

    def body(degp_ref, x_ref, dis_ref, xt_ref):
        deg = jnp.sum(degp_ref[...], axis=0)
        dis = jnp.where(deg > 0, lax.rsqrt(jnp.maximum(deg, 1e-12)), 0.0)
        dis_ref[...] = dis
        xt_ref[...] = dis[:, None] * x_ref[...]

    return pl.pallas_call(
        body,
        grid=(pl.cdiv(n_pad, B),),
        in_specs=[
            pl.BlockSpec((nw, B), lambda i: (0, i)),
            pl.BlockSpec((B, L), lambda i: (i, 0)),
        ],
        out_specs=[
            pl.BlockSpec((B,), lambda i: (i,)),
            pl.BlockSpec((B, L), lambda i: (i, 0)),
        ],
        out_shape=[
            jax.ShapeDtypeStruct((n_pad,), jnp.float32),
            jax.ShapeDtypeStruct((n_pad, L), jnp.float32),
        ],
    )


def _tc_head(n_pad, F):
    B = 2048

    def body(x_ref, accp_ref, dis_ref, wz0, wz1, wh0, wh1, bz, bh, wl, blr, out_ref):
        x = x_ref[...]
        tx1 = -dis_ref[...][:, None] * (accp_ref[0] + accp_ref[1])
        z = jax.nn.sigmoid(
            jnp.dot(x, wz0[...], preferred_element_type=jnp.float32)
            + jnp.dot(tx1, wz1[...], preferred_element_type=jnp.float32)
            + bz[...]
        )
        ht = jnp.tanh(
            jnp.dot(x, wh0[...], preferred_element_type=jnp.float32)
            + jnp.dot(tx1, wh1[...], preferred_element_type=jnp.float32)
            + bh[...]
        )
        h = jax.nn.relu((1.0 - z) * ht)
        o = jnp.dot(h, wl[...], preferred_element_type=jnp.float32) + blr[...]
        out_ref[...] = o[:, 0]

    seq = L
    return pl.pallas_call(
        body,
        grid=(pl.cdiv(n_pad, B),),
        in_specs=[
            pl.BlockSpec((B, seq), lambda i: (i, 0)),
            pl.BlockSpec((NC, B, seq), lambda i: (0, i, 0)),
            pl.BlockSpec((B,), lambda i: (i,)),
            pl.BlockSpec((seq, F), lambda i: (0, 0)),
            pl.BlockSpec((seq, F), lambda i: (0, 0)),
            pl.BlockSpec((seq, F), lambda i: (0, 0)),
            pl.BlockSpec((seq, F), lambda i: (0, 0)),
            pl.BlockSpec((F,), lambda i: (0,)),
            pl.BlockSpec((F,), lambda i: (0,)),
            pl.BlockSpec((F, 1), lambda i: (0, 0)),
            pl.BlockSpec((1, 1), lambda i: (0, 0)),
        ],
        out_specs=pl.BlockSpec((B,), lambda i: (i,)),
        out_shape=jax.ShapeDtypeStruct((n_pad,), jnp.float32),
    )


def kernel(y, edge_index, edge_attr, Wxz0, Wxz1, bxz, Whz0, Whz1, bhz,
           Wxr0, Wxr1, bxr, Whr0, Whr1, bhr, Wxh0, Wxh1, bxh, Whh0, Whh1, bhh,
           Wl, bl):
    n = y.shape[1]
    e = edge_index.shape[1]
    seq = y.shape[0] - 1
    F = Wxz0.shape[1]
    G = 8
    unit = NW * G * 128 * 2
    epad = ((e + unit - 1) // unit) * unit
    R = epad // 128
    n_pad = ((n + 127) // 128) * 128

    src = jnp.pad(edge_index[0], (0, epad - e)).reshape(R, 128)
    dst = jnp.pad(edge_index[1], (0, epad - e)).reshape(R, 128)
    w = jnp.pad(edge_attr[:, 0], (0, epad - e)).reshape(R, 128)
    x = jnp.pad(y[:seq].T, ((0, n_pad - n), (0, 0)))

    degp3 = _deg_kernel(R, G, n_pad)(src, dst, w)
    degp = degp3.reshape(NW, n_pad)
    dis, xt = _tc_mid(n_pad, NW)(degp, x)
    accp = _main_kernel(R, G, n_pad)(src, dst, w, xt)
    out = _tc_head(n_pad, F)(
        x, accp, dis, Wxz0, Wxz1, Wxh0, Wxh1, bxz + bhz, bxh + bhh, Wl,
        bl.reshape(1, 1)
    )
    return out[:n]


# speedup vs baseline: 1.4253x; 1.1403x over previous
import functools

import jax
import jax.numpy as jnp
from jax import lax
from jax.experimental import pallas as pl
from jax.experimental.pallas import tpu as pltpu
from jax.experimental.pallas import tpu_sc as plsc

NC = 2
NS = 16
L = 16
NW = NC * NS


def _sc_mesh():
    return plsc.VectorSubcoreMesh(
        core_axis_name="c", subcore_axis_name="s", num_cores=NC, num_subcores=NS
    )


def _deg_kernel(R, G, n_pad):
    CH = R // (NW * G)
    nr = n_pad // L

    @functools.partial(
        pl.kernel,
        mesh=_sc_mesh(),
        out_type=jax.ShapeDtypeStruct((NW, nr, L), jnp.float32),
        compiler_params=pltpu.CompilerParams(needs_layout_passes=False, use_tc_tiling_on_sc=False),
        scratch_types=[
            pltpu.VMEM((G, 128), jnp.int32),
            pltpu.VMEM((G, 128), jnp.int32),
            pltpu.VMEM((G, 128), jnp.float32),
            pltpu.VMEM((nr, L), jnp.float32),
        ],
    )
    def k(src_h, dst_h, w_h, degp_h, sv, dv, wv, deg_l):
        cid = lax.axis_index("c")
        sid = lax.axis_index("s")
        wid = sid * NC + cid
        zeros = jnp.zeros((L,), jnp.float32)

        def zb(i, carry):
            deg_l[i, :] = zeros
            return carry

        lax.fori_loop(0, nr, zb, 0)

        base = wid * (G * CH)

        def chunk(kc, carry):
            r0 = pl.multiple_of(base + kc * G, 8)
            pltpu.sync_copy(src_h.at[pl.ds(r0, G)], sv)
            pltpu.sync_copy(dst_h.at[pl.ds(r0, G)], dv)
            pltpu.sync_copy(w_h.at[pl.ds(r0, G)], wv)
            for j in range(G * (128 // L)):
                jr, jc = divmod(j, 128 // L)
                s16 = sv[jr, pl.ds(jc * L, L)]
                d16 = dv[jr, pl.ds(jc * L, L)]
                w16 = wv[jr, pl.ds(jc * L, L)]
                row = lax.shift_right_logical(s16, 4)
                col = jnp.bitwise_and(s16, 15)
                plsc.addupdate_scatter(
                    deg_l, [row, col], jnp.where(s16 != d16, w16, 0.0)
                )
            return carry

        lax.fori_loop(0, CH, chunk, 0)
        pltpu.sync_copy(deg_l, degp_h.at[wid])

    return k


def _main_kernel(R, G, n_pad):
    CH = R // (NW * G)
    rpt = n_pad // NS
    ZR = 512
    nfull = rpt // ZR
    rem = rpt % ZR

    @functools.partial(
        pl.kernel,
        mesh=_sc_mesh(),
        out_type=jax.ShapeDtypeStruct((NC, n_pad, L), jnp.float32),
        compiler_params=pltpu.CompilerParams(needs_layout_passes=False, use_tc_tiling_on_sc=False),
        scratch_types=[
            pltpu.VMEM((G, 128), jnp.int32),
            pltpu.VMEM((G, 128), jnp.int32),
            pltpu.VMEM((G, 128), jnp.float32),
            pltpu.VMEM((2, 128, L), jnp.float32),
            pltpu.VMEM((ZR, L), jnp.float32),
            pltpu.VMEM_SHARED((n_pad, L), jnp.float32),
            pltpu.SemaphoreType.DMA,
            pltpu.SemaphoreType.DMA,
        ],
    )
    def k(src_h, dst_h, w_h, xt_h, accp_h, sv, dv, wv, rows, zbuf, acc_sh,
          gsem, ssem):
        cid = lax.axis_index("c")
        sid = lax.axis_index("s")
        wid = sid * NC + cid
        zeros = jnp.zeros((L,), jnp.float32)

        def z1(i, carry):
            zbuf[i, :] = zeros
            return carry

        lax.fori_loop(0, ZR, z1, 0)

        tbase = sid * rpt
        for i in range(nfull):
            pltpu.sync_copy(
                zbuf, acc_sh.at[pl.ds(pl.multiple_of(tbase + i * ZR, 16), ZR)]
            )
        if rem:
            pltpu.sync_copy(
                zbuf.at[pl.ds(0, rem)],
                acc_sh.at[pl.ds(pl.multiple_of(tbase + nfull * ZR, 16), rem)],
            )
        plsc.subcore_barrier()

        base = wid * (G * CH)

        def scale(b, jr):
            for jc in range(128 // L):
                s16 = sv[jr, pl.ds(jc * L, L)]
                d16 = dv[jr, pl.ds(jc * L, L)]
                w16 = wv[jr, pl.ds(jc * L, L)]
                wp = jnp.where(s16 != d16, w16, 0.0)
                for k in range(L):
                    e = jc * L + k
                    rows[b, e, :] = rows[b, e, :] * wp[k]

        def chunk(kc, carry):
            r0 = pl.multiple_of(base + kc * G, 8)
            pltpu.sync_copy(src_h.at[pl.ds(r0, G)], sv)
            pltpu.sync_copy(dst_h.at[pl.ds(r0, G)], dv)
            pltpu.sync_copy(w_h.at[pl.ds(r0, G)], wv)
            gh = [None] * G
            sh = [None] * G
            gh[0] = pltpu.async_copy(xt_h.at[sv.at[0]], rows.at[0], gsem)
            for jr in range(G):
                b = jr & 1
                if jr >= 1:
                    sh[jr - 1].wait()
                gh[jr].wait()
                if jr + 1 < G:
                    gh[jr + 1] = pltpu.async_copy(
                        xt_h.at[sv.at[jr + 1]], rows.at[1 - b], gsem
                    )
                scale(b, jr)
                sh[jr] = pltpu.async_copy(
                    rows.at[b], acc_sh.at[dv.at[jr]], ssem, add=True
                )
            sh[G - 1].wait()
            return carry

        lax.fori_loop(0, CH, chunk, 0)
        plsc.subcore_barrier()
        r0 = pl.multiple_of(sid * rpt, 16)
        pltpu.sync_copy(
            acc_sh.at[pl.ds(r0, rpt)], accp_h.at[cid, pl.ds(r0, rpt)]
        )

    return k


def _tc_mid(n_pad, nw):
    B = 2048

    def body(degp_ref, x_ref, dis_ref, xt_ref):
        deg = jnp.sum(degp_ref[...], axis=0)
        dis = jnp.where(deg > 0, lax.rsqrt(jnp.maximum(deg, 1e-12)), 0.0)
        dis_ref[...] = dis
        xt_ref[...] = dis[:, None] * x_ref[...]

    return pl.pallas_call(
        body,
        grid=(pl.cdiv(n_pad, B),),
        in_specs=[
            pl.BlockSpec((nw, B), lambda i: (0, i)),
            pl.BlockSpec((B, L), lambda i: (i, 0)),
        ],
        out_specs=[
            pl.BlockSpec((B,), lambda i: (i,)),
            pl.BlockSpec((B, L), lambda i: (i, 0)),
        ],
        out_shape=[
            jax.ShapeDtypeStruct((n_pad,), jnp.float32),
            jax.ShapeDtypeStruct((n_pad, L), jnp.float32),
        ],
    )


def _tc_head(n_pad, F):
    B = 2048

    def body(x_ref, accp_ref, dis_ref, wz0, wz1, wh0, wh1, bz, bh, wl, blr, out_ref):
        x = x_ref[...]
        tx1 = -dis_ref[...][:, None] * (accp_ref[0] + accp_ref[1])
        z = jax.nn.sigmoid(
            jnp.dot(x, wz0[...], preferred_element_type=jnp.float32)
            + jnp.dot(tx1, wz1[...], preferred_element_type=jnp.float32)
            + bz[...]
        )
        ht = jnp.tanh(
            jnp.dot(x, wh0[...], preferred_element_type=jnp.float32)
            + jnp.dot(tx1, wh1[...], preferred_element_type=jnp.float32)
            + bh[...]
        )
        h = jax.nn.relu((1.0 - z) * ht)
        o = jnp.dot(h, wl[...], preferred_element_type=jnp.float32) + blr[...]
        out_ref[...] = o[:, 0]

    seq = L
    return pl.pallas_call(
        body,
        grid=(pl.cdiv(n_pad, B),),
        in_specs=[
            pl.BlockSpec((B, seq), lambda i: (i, 0)),
            pl.BlockSpec((NC, B, seq), lambda i: (0, i, 0)),
            pl.BlockSpec((B,), lambda i: (i,)),
            pl.BlockSpec((seq, F), lambda i: (0, 0)),
            pl.BlockSpec((seq, F), lambda i: (0, 0)),
            pl.BlockSpec((seq, F), lambda i: (0, 0)),
            pl.BlockSpec((seq, F), lambda i: (0, 0)),
            pl.BlockSpec((F,), lambda i: (0,)),
            pl.BlockSpec((F,), lambda i: (0,)),
            pl.BlockSpec((F, 1), lambda i: (0, 0)),
            pl.BlockSpec((1, 1), lambda i: (0, 0)),
        ],
        out_specs=pl.BlockSpec((B,), lambda i: (i,)),
        out_shape=jax.ShapeDtypeStruct((n_pad,), jnp.float32),
    )


def kernel(y, edge_index, edge_attr, Wxz0, Wxz1, bxz, Whz0, Whz1, bhz,
           Wxr0, Wxr1, bxr, Whr0, Whr1, bhr, Wxh0, Wxh1, bxh, Whh0, Whh1, bhh,
           Wl, bl):
    n = y.shape[1]
    e = edge_index.shape[1]
    seq = y.shape[0] - 1
    F = Wxz0.shape[1]
    G = 8
    unit = NW * G * 128 * 2
    epad = ((e + unit - 1) // unit) * unit
    R = epad // 128
    n_pad = ((n + 127) // 128) * 128

    src = jnp.pad(edge_index[0], (0, epad - e)).reshape(R, 128)
    dst = jnp.pad(edge_index[1], (0, epad - e)).reshape(R, 128)
    w = jnp.pad(edge_attr[:, 0], (0, epad - e)).reshape(R, 128)
    x = jnp.pad(y[:seq].T, ((0, n_pad - n), (0, 0)))

    degp3 = _deg_kernel(R, G, n_pad)(src, dst, w)
    degp = degp3.reshape(NW, n_pad)
    dis, xt = _tc_mid(n_pad, NW)(degp, x)
    accp = _main_kernel(R, G, n_pad)(src, dst, w, xt)
    out = _tc_head(n_pad, F)(
        x, accp, dis, Wxz0, Wxz1, Wxh0, Wxh1, bxz + bhz, bxh + bhh, Wl,
        bl.reshape(1, 1)
    )
    return out[:n]
